# baseline (speedup 1.0000x reference)
import functools

import jax
import jax.numpy as jnp
from jax.experimental import pallas as pl

TOKENS = 8192
M = 2048
T = 64
K = 2
BLK = 2048


def _router_block(x_ref, w_ref, b_ref, noise_ref, out_ref, idx_ref):
    xb = x_ref[...]
    w = w_ref[...]
    acc = jnp.dot(xb.astype(jnp.bfloat16), w.astype(jnp.bfloat16),
                  preferred_element_type=jnp.float32) + b_ref[...]
    logits = acc[:, :T]
    pre = acc[:, T:]
    ns = jnp.maximum(pre, 0.0) + jnp.log1p(jnp.exp(-jnp.abs(pre)))
    h = logits + noise_ref[...] * ns

    iota = jax.lax.broadcasted_iota(jnp.int32, (BLK, T), 1)
    v1 = jnp.max(h, axis=-1, keepdims=True)
    i1 = jnp.min(jnp.where(h == v1, iota, T), axis=-1, keepdims=True)
    h2 = jnp.where(iota == i1, -jnp.inf, h)
    v2 = jnp.max(h2, axis=-1, keepdims=True)
    i2 = jnp.min(jnp.where(h2 == v2, iota, T), axis=-1, keepdims=True)

    e2 = jnp.exp(v2 - v1)
    denom = 1.0 + e2
    p1 = 1.0 / denom
    p2 = e2 / denom
    gates = jnp.where(iota == i1, p1, jnp.where(iota == i2, p2, 0.0))

    out_ref[...] = jnp.concatenate([gates, h, ns, logits], axis=1)
    idx_ref[...] = jnp.concatenate([i1, i2], axis=1)


_NOISE_CACHE = []


def _noise_const():
    if not _NOISE_CACHE:
        _NOISE_CACHE.append(jax.random.normal(
            jax.random.key(42), (TOKENS, T), dtype=jnp.float32))
    return _NOISE_CACHE[0]


@functools.partial(jax.jit, static_argnums=())
def kernel(x, Wg_w, Wg_b, Wn_w, Wn_b):
    w = jnp.concatenate([Wg_w, Wn_w], axis=0).T
    b = jnp.concatenate([Wg_b, Wn_b], axis=0)[None, :]
    noise = _noise_const()
    grid = (TOKENS // BLK,)
    fused, topk_idx = pl.pallas_call(
        _router_block,
        grid=grid,
        in_specs=[
            pl.BlockSpec((BLK, M), lambda i: (i, 0)),
            pl.BlockSpec((M, 2 * T), lambda i: (0, 0)),
            pl.BlockSpec((1, 2 * T), lambda i: (0, 0)),
            pl.BlockSpec((BLK, T), lambda i: (i, 0)),
        ],
        out_specs=[
            pl.BlockSpec((BLK, 4 * T), lambda i: (i, 0)),
            pl.BlockSpec((BLK, K), lambda i: (i, 0)),
        ],
        out_shape=[
            jax.ShapeDtypeStruct((TOKENS, 4 * T), jnp.float32),
            jax.ShapeDtypeStruct((TOKENS, K), jnp.int32),
        ],
    )(x, w, b, noise)
    gates = fused[:, :T]
    h = fused[:, T:2 * T]
    noise_scale = fused[:, 2 * T:3 * T]
    logits = fused[:, 3 * T:]
    return (gates, h, topk_idx, noise_scale, logits)


# speedup vs baseline: 1.2006x; 1.2006x over previous
import functools

import jax
import jax.numpy as jnp
from jax.experimental import pallas as pl

TOKENS = 8192
M = 2048
T = 64
K = 2
BLK = 1024


def _router_block(x_ref, w_ref, b_ref, noise_ref,
                  gates_ref, h_ref, idx_ref, ns_ref, logits_ref):
    r0 = pl.program_id(0) * BLK
    xb = x_ref[...]
    w = w_ref[...]
    acc = jnp.dot(xb.astype(jnp.bfloat16), w.astype(jnp.bfloat16),
                  preferred_element_type=jnp.float32) + b_ref[...]
    logits = acc[:, :T]
    pre = acc[:, T:]
    ns = jnp.maximum(pre, 0.0) + jnp.log1p(jnp.exp(-jnp.abs(pre)))
    h = logits + noise_ref[pl.ds(r0, BLK), :] * ns

    iota = jax.lax.broadcasted_iota(jnp.int32, (BLK, T), 1)
    v1 = jnp.max(h, axis=-1, keepdims=True)
    i1 = jnp.min(jnp.where(h == v1, iota, T), axis=-1, keepdims=True)
    h2 = jnp.where(iota == i1, -jnp.inf, h)
    v2 = jnp.max(h2, axis=-1, keepdims=True)
    i2 = jnp.min(jnp.where(h2 == v2, iota, T), axis=-1, keepdims=True)

    e2 = jnp.exp(v2 - v1)
    denom = 1.0 + e2
    p1 = 1.0 / denom
    p2 = e2 / denom
    gates = jnp.where(iota == i1, p1, jnp.where(iota == i2, p2, 0.0))

    gates_ref[pl.ds(r0, BLK), :] = gates
    h_ref[pl.ds(r0, BLK), :] = h
    idx_ref[pl.ds(r0, BLK), :] = jnp.concatenate([i1, i2], axis=1)
    ns_ref[pl.ds(r0, BLK), :] = ns
    logits_ref[pl.ds(r0, BLK), :] = logits


_NOISE_CACHE = []


def _noise_const():
    if not _NOISE_CACHE:
        _NOISE_CACHE.append(jax.random.normal(
            jax.random.key(42), (TOKENS, T), dtype=jnp.float32))
    return _NOISE_CACHE[0]


@functools.partial(jax.jit, static_argnums=())
def kernel(x, Wg_w, Wg_b, Wn_w, Wn_b):
    w = jnp.concatenate([Wg_w, Wn_w], axis=0).T
    b = jnp.concatenate([Wg_b, Wn_b], axis=0)[None, :]
    noise = _noise_const()
    grid = (TOKENS // BLK,)
    whole = lambda shape: pl.BlockSpec(shape, lambda i: tuple(0 for _ in shape))
    out = pl.pallas_call(
        _router_block,
        grid=grid,
        in_specs=[
            pl.BlockSpec((BLK, M), lambda i: (i, 0)),
            whole((M, 2 * T)),
            whole((1, 2 * T)),
            whole((TOKENS, T)),
        ],
        out_specs=[
            whole((TOKENS, T)),
            whole((TOKENS, T)),
            whole((TOKENS, K)),
            whole((TOKENS, T)),
            whole((TOKENS, T)),
        ],
        out_shape=[
            jax.ShapeDtypeStruct((TOKENS, T), jnp.float32),
            jax.ShapeDtypeStruct((TOKENS, T), jnp.float32),
            jax.ShapeDtypeStruct((TOKENS, K), jnp.int32),
            jax.ShapeDtypeStruct((TOKENS, T), jnp.float32),
            jax.ShapeDtypeStruct((TOKENS, T), jnp.float32),
        ],
    )(x, w, b, noise)
    gates, h, topk_idx, noise_scale, logits = out
    return (gates, h, topk_idx, noise_scale, logits)


# speedup vs baseline: 1.2227x; 1.0184x over previous
import functools

import jax
import jax.numpy as jnp
from jax.experimental import pallas as pl
from jax.experimental.pallas import tpu as pltpu

TOKENS = 8192
M = 2048
T = 64
K = 2
BLK = 1024
NBLK = TOKENS // BLK


def _router_block(x_ref, w_ref, b_ref, noise_ref,
                  gates_ref, h_ref, idx_ref, ns_ref, logits_ref, acc_ref):
    i = pl.program_id(0)

    @pl.when(i > 0)
    def _route():
        r0 = (i - 1) * BLK
        a = acc_ref[...]
        logits = a[:, :T]
        pre = a[:, T:]
        ns = jnp.maximum(pre, 0.0) + jnp.log1p(jnp.exp(-jnp.abs(pre)))
        h = logits + noise_ref[pl.ds(r0, BLK), :] * ns

        iota = jax.lax.broadcasted_iota(jnp.int32, (BLK, T), 1)
        v1 = jnp.max(h, axis=-1, keepdims=True)
        i1 = jnp.min(jnp.where(h == v1, iota, T), axis=-1, keepdims=True)
        h2 = jnp.where(iota == i1, -jnp.inf, h)
        v2 = jnp.max(h2, axis=-1, keepdims=True)
        i2 = jnp.min(jnp.where(h2 == v2, iota, T), axis=-1, keepdims=True)

        e2 = jnp.exp(v2 - v1)
        denom = 1.0 + e2
        p1 = 1.0 / denom
        p2 = e2 / denom
        gates = jnp.where(iota == i1, p1, jnp.where(iota == i2, p2, 0.0))

        gates_ref[pl.ds(r0, BLK), :] = gates
        h_ref[pl.ds(r0, BLK), :] = h
        idx_ref[pl.ds(r0, BLK), :] = jnp.concatenate([i1, i2], axis=1)
        ns_ref[pl.ds(r0, BLK), :] = ns
        logits_ref[pl.ds(r0, BLK), :] = logits

    @pl.when(i < NBLK)
    def _matmul():
        xb = x_ref[...]
        w = w_ref[...]
        acc_ref[...] = jnp.dot(
            xb.astype(jnp.bfloat16), w.astype(jnp.bfloat16),
            preferred_element_type=jnp.float32) + b_ref[...]


_NOISE_CACHE = []


def _noise_const():
    if not _NOISE_CACHE:
        _NOISE_CACHE.append(jax.random.normal(
            jax.random.key(42), (TOKENS, T), dtype=jnp.float32))
    return _NOISE_CACHE[0]


@functools.partial(jax.jit, static_argnums=())
def kernel(x, Wg_w, Wg_b, Wn_w, Wn_b):
    w = jnp.concatenate([Wg_w, Wn_w], axis=0).T
    b = jnp.concatenate([Wg_b, Wn_b], axis=0)[None, :]
    noise = _noise_const()
    grid = (NBLK + 1,)
    whole = lambda shape: pl.BlockSpec(shape, lambda i: tuple(0 for _ in shape))
    out = pl.pallas_call(
        _router_block,
        grid=grid,
        in_specs=[
            pl.BlockSpec((BLK, M), lambda i: (jnp.minimum(i, NBLK - 1), 0)),
            whole((M, 2 * T)),
            whole((1, 2 * T)),
            whole((TOKENS, T)),
        ],
        out_specs=[
            whole((TOKENS, T)),
            whole((TOKENS, T)),
            whole((TOKENS, K)),
            whole((TOKENS, T)),
            whole((TOKENS, T)),
        ],
        out_shape=[
            jax.ShapeDtypeStruct((TOKENS, T), jnp.float32),
            jax.ShapeDtypeStruct((TOKENS, T), jnp.float32),
            jax.ShapeDtypeStruct((TOKENS, K), jnp.int32),
            jax.ShapeDtypeStruct((TOKENS, T), jnp.float32),
            jax.ShapeDtypeStruct((TOKENS, T), jnp.float32),
        ],
        scratch_shapes=[pltpu.VMEM((BLK, 2 * T), jnp.float32)],
    )(x, w, b, noise)
    gates, h, topk_idx, noise_scale, logits = out
    return (gates, h, topk_idx, noise_scale, logits)


# speedup vs baseline: 1.2385x; 1.0129x over previous
import functools

import jax
import jax.numpy as jnp
from jax.experimental import pallas as pl

TOKENS = 8192
M = 2048
T = 64
K = 2
BLK = 1024


def _router_block(x_ref, w_ref, b_ref, noise_ref,
                  gates_ref, h_ref, idx_ref, ns_ref, logits_ref):
    r0 = pl.program_id(0) * BLK
    xb = x_ref[...]
    w = w_ref[...]
    acc = jnp.dot(xb.astype(jnp.bfloat16), w.astype(jnp.bfloat16),
                  preferred_element_type=jnp.float32) + b_ref[...]
    logits = acc[:, :T]
    pre = acc[:, T:]
    ns = jnp.maximum(pre, 0.0) + jnp.log1p(jnp.exp(-jnp.abs(pre)))
    h = logits + noise_ref[pl.ds(r0, BLK), :] * ns

    iota = jax.lax.broadcasted_iota(jnp.int32, (BLK, T), 1)
    v1 = jnp.max(h, axis=-1, keepdims=True)
    i1 = jnp.min(jnp.where(h == v1, iota, T), axis=-1, keepdims=True)
    h2 = jnp.where(iota == i1, -jnp.inf, h)
    v2 = jnp.max(h2, axis=-1, keepdims=True)
    i2 = jnp.min(jnp.where(h2 == v2, iota, T), axis=-1, keepdims=True)

    e2 = jnp.exp(v2 - v1)
    denom = 1.0 + e2
    p1 = 1.0 / denom
    p2 = e2 / denom
    gates = jnp.where(iota == i1, p1, jnp.where(iota == i2, p2, 0.0))

    gates_ref[...] = gates
    h_ref[...] = h
    idx_ref[...] = jnp.concatenate([i1, i2], axis=1)
    ns_ref[...] = ns
    logits_ref[...] = logits


_NOISE_CACHE = []


def _noise_const():
    if not _NOISE_CACHE:
        _NOISE_CACHE.append(jax.random.normal(
            jax.random.key(42), (TOKENS, T), dtype=jnp.float32))
    return _NOISE_CACHE[0]


@functools.partial(jax.jit, static_argnums=())
def kernel(x, Wg_w, Wg_b, Wn_w, Wn_b):
    w = jnp.concatenate([Wg_w, Wn_w], axis=0).T
    b = jnp.concatenate([Wg_b, Wn_b], axis=0)[None, :]
    noise = _noise_const()
    grid = (TOKENS // BLK,)
    out = pl.pallas_call(
        _router_block,
        grid=grid,
        in_specs=[
            pl.BlockSpec((BLK, M), lambda i: (i, 0)),
            pl.BlockSpec((M, 2 * T), lambda i: (0, 0)),
            pl.BlockSpec((1, 2 * T), lambda i: (0, 0)),
            pl.BlockSpec((TOKENS, T), lambda i: (0, 0)),
        ],
        out_specs=[
            pl.BlockSpec((BLK, T), lambda i: (i, 0)),
            pl.BlockSpec((BLK, T), lambda i: (i, 0)),
            pl.BlockSpec((BLK, K), lambda i: (i, 0)),
            pl.BlockSpec((BLK, T), lambda i: (i, 0)),
            pl.BlockSpec((BLK, T), lambda i: (i, 0)),
        ],
        out_shape=[
            jax.ShapeDtypeStruct((TOKENS, T), jnp.float32),
            jax.ShapeDtypeStruct((TOKENS, T), jnp.float32),
            jax.ShapeDtypeStruct((TOKENS, K), jnp.int32),
            jax.ShapeDtypeStruct((TOKENS, T), jnp.float32),
            jax.ShapeDtypeStruct((TOKENS, T), jnp.float32),
        ],
    )(x, w, b, noise)
    gates, h, topk_idx, noise_scale, logits = out
    return (gates, h, topk_idx, noise_scale, logits)


# speedup vs baseline: 1.3357x; 1.0785x over previous
import functools

import jax
import jax.numpy as jnp
from jax import lax
from jax.experimental import pallas as pl
from jax.experimental.pallas import tpu as pltpu
from jax.experimental.pallas import tpu_sc as plsc

TOKENS = 8192
M = 2048
T = 64
K = 2
BLK = 1024
LANES = 16

_SC_INFO = plsc.get_sparse_core_info()
NC = _SC_INFO.num_cores
NS = _SC_INFO.num_subcores
NW = NC * NS
COLS_PER_W = TOKENS // NW


def _mm_softplus_block(x_ref, w_ref, b_ref,
                       logits_ref, ns_ref, lot_ref, nst_ref):
    xb = x_ref[...]
    w = w_ref[...]
    acc = jnp.dot(xb.astype(jnp.bfloat16), w.astype(jnp.bfloat16),
                  preferred_element_type=jnp.float32) + b_ref[...]
    logits = acc[:, :T]
    pre = acc[:, T:]
    ns = jnp.maximum(pre, 0.0) + jnp.log1p(jnp.exp(-jnp.abs(pre)))
    logits_ref[...] = logits
    ns_ref[...] = ns
    lot_ref[...] = logits.T
    nst_ref[...] = ns.T


def _tc_logits_ns(x, w, b):
    return pl.pallas_call(
        _mm_softplus_block,
        grid=(TOKENS // BLK,),
        in_specs=[
            pl.BlockSpec((BLK, M), lambda i: (i, 0)),
            pl.BlockSpec((M, 2 * T), lambda i: (0, 0)),
            pl.BlockSpec((1, 2 * T), lambda i: (0, 0)),
        ],
        out_specs=[
            pl.BlockSpec((BLK, T), lambda i: (i, 0)),
            pl.BlockSpec((BLK, T), lambda i: (i, 0)),
            pl.BlockSpec((T, BLK), lambda i: (0, i)),
            pl.BlockSpec((T, BLK), lambda i: (0, i)),
        ],
        out_shape=[
            jax.ShapeDtypeStruct((TOKENS, T), jnp.float32),
            jax.ShapeDtypeStruct((TOKENS, T), jnp.float32),
            jax.ShapeDtypeStruct((T, TOKENS), jnp.float32),
            jax.ShapeDtypeStruct((T, TOKENS), jnp.float32),
        ],
    )(x, w, b)


def _sc_route(lo_t, ns_t, nz_t):
    mesh = plsc.VectorSubcoreMesh(core_axis_name="c", subcore_axis_name="s")

    @functools.partial(
        pl.kernel,
        mesh=mesh,
        out_type=[
            jax.ShapeDtypeStruct((T, TOKENS), jnp.float32),
            jax.ShapeDtypeStruct((T, TOKENS), jnp.float32),
            jax.ShapeDtypeStruct((K, TOKENS), jnp.int32),
        ],
        scratch_types=[
            pltpu.VMEM((T, COLS_PER_W), jnp.float32),
            pltpu.VMEM((T, COLS_PER_W), jnp.float32),
            pltpu.VMEM((T, COLS_PER_W), jnp.float32),
            pltpu.VMEM((T, COLS_PER_W), jnp.float32),
            pltpu.VMEM((T, COLS_PER_W), jnp.float32),
            pltpu.VMEM((K, COLS_PER_W), jnp.int32),
        ],
    )
    def route(lot_hbm, nst_hbm, nzt_hbm, gt_hbm, ht_hbm, ixt_hbm,
              lo_v, ns_v, nz_v, g_v, h_v, ix_v):
        wid = lax.axis_index("s") * NC + lax.axis_index("c")
        base = wid * COLS_PER_W
        csl = pl.ds(base, COLS_PER_W)
        pltpu.sync_copy(lot_hbm.at[:, csl], lo_v)
        pltpu.sync_copy(nst_hbm.at[:, csl], ns_v)
        pltpu.sync_copy(nzt_hbm.at[:, csl], nz_v)

        def group(g, _):
            col = pl.ds(g * LANES, LANES)
            v1 = jnp.full((LANES,), -jnp.inf, jnp.float32)
            v2 = jnp.full((LANES,), -jnp.inf, jnp.float32)
            i1 = jnp.zeros((LANES,), jnp.int32)
            i2 = jnp.zeros((LANES,), jnp.int32)
            for e in range(T):
                he = lo_v[e, col] + nz_v[e, col] * ns_v[e, col]
                h_v[e, col] = he
                new1 = he > v1
                gt2 = he > v2
                v2 = jnp.where(new1, v1, jnp.where(gt2, he, v2))
                i2 = jnp.where(new1, i1, jnp.where(gt2, e, i2))
                v1 = jnp.where(new1, he, v1)
                i1 = jnp.where(new1, e, i1)

            e2 = jnp.exp(v2 - v1)
            denom = 1.0 + e2
            p1 = 1.0 / denom
            p2 = e2 / denom
            zero = jnp.zeros((LANES,), jnp.float32)
            for e in range(T):
                g_v[e, col] = jnp.where(i1 == e, p1,
                                        jnp.where(i2 == e, p2, zero))
            ix_v[0, col] = i1
            ix_v[1, col] = i2
            return 0

        lax.fori_loop(0, COLS_PER_W // LANES, group, 0)

        pltpu.sync_copy(g_v, gt_hbm.at[:, csl])
        pltpu.sync_copy(h_v, ht_hbm.at[:, csl])
        pltpu.sync_copy(ix_v, ixt_hbm.at[:, csl])

    return route(lo_t, ns_t, nz_t)


_NOISE_CACHE = []


def _noise_const_t():
    if not _NOISE_CACHE:
        _NOISE_CACHE.append(jax.random.normal(
            jax.random.key(42), (TOKENS, T), dtype=jnp.float32).T)
    return _NOISE_CACHE[0]


@functools.partial(jax.jit, static_argnums=())
def kernel(x, Wg_w, Wg_b, Wn_w, Wn_b):
    w = jnp.concatenate([Wg_w, Wn_w], axis=0).T
    b = jnp.concatenate([Wg_b, Wn_b], axis=0)[None, :]
    noise_t = _noise_const_t()
    logits, noise_scale, lo_t, ns_t = _tc_logits_ns(x, w, b)
    gates_t, h_t, idx_t = _sc_route(lo_t, ns_t, noise_t)
    return (gates_t.T, h_t.T, idx_t.T, noise_scale, logits)
